# Initial kernel scaffold; baseline (speedup 1.0000x reference)
import jax
import jax.numpy as jnp
from jax.experimental import pallas as pl


def kernel(x, edge_index, W1, b1, g1, be1, W2, b2, g2, be2, W3, b3, g3, be3):
    raise NotImplementedError("write your pallas kernel here")


# speedup vs baseline: 28.0009x; 28.0009x over previous
import functools

import jax
import jax.numpy as jnp
import numpy as np
from jax import lax
from jax.experimental import pallas as pl
from jax.experimental.pallas import tpu as pltpu
from jax.experimental.pallas import tpu_sc as plsc

NC = 2
NS = 16
NW = NC * NS
K = 128
NBUF = 4


def _mesh():
    return plsc.VectorSubcoreMesh(
        core_axis_name="c", subcore_axis_name="s", num_cores=NC, num_subcores=NS
    )


def _flush_rows(n):
    return ((n + 1 + 16 * NS - 1) // (16 * NS)) * 16


def _zero_fill_2d(ref, rows, cols):
    z16 = jnp.zeros((16,), jnp.float32)

    @pl.loop(0, rows)
    def _(i):
        for c in range(cols // 16):
            ref[i, pl.ds(c * 16, 16)] = z16


def _make_deg_kernel(n, nch):
    fl = _flush_rows(n)
    nacc = NS * fl

    @functools.partial(
        pl.kernel,
        out_type=jax.ShapeDtypeStruct((NC, NS, fl), jnp.float32),
        mesh=_mesh(),
        scratch_types=[
            pltpu.VMEM((nch, K), jnp.int32),
            pltpu.VMEM((K,), jnp.float32),
            pltpu.VMEM((fl,), jnp.float32),
            pltpu.VMEM_SHARED((nacc,), jnp.float32),
        ],
        compiler_params=pltpu.CompilerParams(use_tc_tiling_on_sc=False),
    )
    def deg_k(dst_hbm, out_hbm, dst_v, ones_v, zb_v, acc_sh):
        cid = lax.axis_index("c")
        sid = lax.axis_index("s")
        wid = cid * NS + sid
        pltpu.sync_copy(dst_hbm.at[wid], dst_v)

        one16 = jnp.ones((16,), jnp.float32)
        zero16 = jnp.zeros((16,), jnp.float32)

        @pl.loop(0, K // 16)
        def _(i):
            ones_v[pl.ds(i * 16, 16)] = one16

        @pl.loop(0, fl // 16)
        def _(i):
            zb_v[pl.ds(i * 16, 16)] = zero16

        pltpu.sync_copy(zb_v, acc_sh.at[pl.ds(sid * fl, fl)])
        plsc.subcore_barrier()

        @pl.loop(0, nch)
        def _(j):
            pltpu.sync_copy(ones_v, acc_sh.at[dst_v.at[j]], add=True)

        plsc.subcore_barrier()
        pltpu.sync_copy(acc_sh.at[pl.ds(sid * fl, fl)], zb_v)
        pltpu.sync_copy(zb_v, out_hbm.at[cid, sid])

    return deg_k


def _make_prop_kernel(n, c, nch):
    fl = _flush_rows(n)
    nacc = NS * fl

    @functools.partial(
        pl.kernel,
        out_type=jax.ShapeDtypeStruct((NC, NS, fl, c), jnp.float32),
        mesh=_mesh(),
        scratch_types=[
            pltpu.VMEM((nch, K), jnp.int32),
            pltpu.VMEM((nch, K), jnp.int32),
            pltpu.VMEM((NBUF, K, c), jnp.float32),
            pltpu.VMEM((fl, c), jnp.float32),
            pltpu.VMEM_SHARED((nacc, c), jnp.float32),
        ]
        + [pltpu.SemaphoreType.DMA] * NBUF,
        compiler_params=pltpu.CompilerParams(use_tc_tiling_on_sc=False),
    )
    def prop_k(ht_hbm, src_hbm, dst_hbm, out_hbm, src_v, dst_v, ring_v, zb_v,
               acc_sh, *sems):
        cid = lax.axis_index("c")
        sid = lax.axis_index("s")
        wid = cid * NS + sid
        pltpu.sync_copy(src_hbm.at[wid], src_v)
        pltpu.sync_copy(dst_hbm.at[wid], dst_v)

        _zero_fill_2d(zb_v, fl, c)
        pltpu.sync_copy(zb_v, acc_sh.at[pl.ds(sid * fl, fl)])
        plsc.subcore_barrier()

        def gather(j, b):
            return pltpu.make_async_copy(
                ht_hbm.at[src_v.at[j]], ring_v.at[b], sems[b]
            )

        for b in range(NBUF):
            gather(b, b).start()

        @pl.loop(0, (nch - NBUF) // NBUF)
        def _(t):
            go = t * NBUF
            for b in range(NBUF):
                j = go + b
                gather(j, b).wait()
                pltpu.sync_copy(ring_v.at[b], acc_sh.at[dst_v.at[j]], add=True)
                gather(j + NBUF, b).start()

        for b in range(NBUF):
            j = nch - NBUF + b
            gather(j, b).wait()
            pltpu.sync_copy(ring_v.at[b], acc_sh.at[dst_v.at[j]], add=True)

        plsc.subcore_barrier()
        pltpu.sync_copy(acc_sh.at[pl.ds(sid * fl, fl)], zb_v)
        pltpu.sync_copy(zb_v, out_hbm.at[cid, sid])

    return prop_k


def _tc_call(fn, out_shapes, *args):
    return pl.pallas_call(fn, out_shape=out_shapes)(*args)


def _tc1_body(x_ref, w_ref, d0_ref, d1_ref, ht_ref, dinv_ref):
    deg = d0_ref[...] + d1_ref[...] + 1.0
    dinv = 1.0 / jnp.sqrt(deg)
    h = jnp.dot(x_ref[...], w_ref[...], preferred_element_type=jnp.float32)
    ht_ref[...] = h * dinv
    dinv_ref[...] = dinv


def _bn_act(p, g, be, leaky):
    m = jnp.mean(p, axis=0, keepdims=True)
    d = p - m
    v = jnp.mean(d * d, axis=0, keepdims=True)
    y = d * (g / jnp.sqrt(v + 1e-5)) + be
    if leaky:
        return jnp.where(y >= 0.0, y, 0.01 * y)
    return jnp.maximum(y, 0.0)


def _tcb_body(p0_ref, p1_ref, ht_ref, dinv_ref, g_ref, be_ref, out_ref):
    dinv = dinv_ref[...]
    p = dinv * (p0_ref[...] + p1_ref[...] + ht_ref[...])
    y = _bn_act(p, g_ref[...], be_ref[...], leaky=True)
    out_ref[...] = dinv * y


def _tcc_body(p0_ref, p1_ref, ht_ref, dinv_ref, w_ref, g_ref, be_ref, out_ref):
    dinv = dinv_ref[...]
    p = dinv * (p0_ref[...] + p1_ref[...] + ht_ref[...])
    h = jnp.dot(p, w_ref[...], preferred_element_type=jnp.float32)
    y = _bn_act(h, g_ref[...], be_ref[...], leaky=True)
    out_ref[...] = dinv * y


def _tcd_body(p0_ref, p1_ref, ht_ref, dinv_ref, w_ref, g_ref, be_ref, pool_ref,
              out_ref):
    dinv = dinv_ref[...]
    p = dinv * (p0_ref[...] + p1_ref[...] + ht_ref[...])
    h = jnp.dot(p, w_ref[...], preferred_element_type=jnp.float32)
    y = _bn_act(h, g_ref[...], be_ref[...], leaky=False)
    out_ref[...] = jnp.dot(y, pool_ref[...], preferred_element_type=jnp.float32)


def kernel(x, edge_index, W1, b1, g1, be1, W2, b2, g2, be2, W3, b3, g3, be3):
    del b1, b2, b3
    n = x.shape[0]
    e = edge_index.shape[1]
    src = edge_index[0].astype(jnp.int32)
    dst = edge_index[1].astype(jnp.int32)

    ept = -(-e // NW)
    nch = -(-(-(-ept // K)) // NBUF) * NBUF
    pad = NW * nch * K - e
    srcp = jnp.concatenate([src, jnp.zeros((pad,), jnp.int32)]).reshape(NW, nch, K)
    dstp = jnp.concatenate([dst, jnp.full((pad,), n, jnp.int32)]).reshape(NW, nch, K)

    fl = _flush_rows(n)

    degp = _make_deg_kernel(n, nch)(dstp)
    degp = degp.reshape(NC, NS * fl)[:, :n]
    deg0 = degp[0].reshape(n, 1)
    deg1 = degp[1].reshape(n, 1)

    c1, c2, c3 = W1.shape[1], W2.shape[1], W3.shape[1]
    f32 = jnp.float32

    ht1, dinv = _tc_call(
        _tc1_body,
        [jax.ShapeDtypeStruct((n, c1), f32), jax.ShapeDtypeStruct((n, 1), f32)],
        x, W1, deg0, deg1,
    )

    def prop(ht, c):
        parts = _make_prop_kernel(n, c, nch)(ht, srcp, dstp)
        parts = parts.reshape(NC, NS * fl, c)[:, :n]
        return parts[0], parts[1]

    p0, p1 = prop(ht1, c1)
    ht2 = _tc_call(
        _tcb_body,
        jax.ShapeDtypeStruct((n, c1), f32),
        p0, p1, ht1, dinv, g1.reshape(1, c1), be1.reshape(1, c1),
    )

    p0, p1 = prop(ht2, c1)
    ht3 = _tc_call(
        _tcc_body,
        jax.ShapeDtypeStruct((n, c2), f32),
        p0, p1, ht2, dinv, W2, g2.reshape(1, c2), be2.reshape(1, c2),
    )

    p0, p1 = prop(ht3, c2)
    pool = np.zeros((c3, c3 // 4), np.float32)
    for i in range(c3):
        pool[i, i // 4] = 0.25
    out = _tc_call(
        _tcd_body,
        jax.ShapeDtypeStruct((n, c3 // 4), f32),
        p0, p1, ht3, dinv, W3, g3.reshape(1, c3), be3.reshape(1, c3),
        jnp.asarray(pool),
    )
    return out


# speedup vs baseline: 44.6507x; 1.5946x over previous
import functools

import jax
import jax.numpy as jnp
import numpy as np
from jax import lax
from jax.experimental import pallas as pl
from jax.experimental.pallas import tpu as pltpu
from jax.experimental.pallas import tpu_sc as plsc

NC = 2
NS = 16
NW = NC * NS
K = 128
NBUF = 4


def _mesh():
    return plsc.VectorSubcoreMesh(
        core_axis_name="c", subcore_axis_name="s", num_cores=NC, num_subcores=NS
    )


def _flush_rows(n):
    return ((n + 128 + 16 * NS - 1) // (16 * NS)) * 16


def _zero_fill_2d(ref, rows, cols):
    z16 = jnp.zeros((16,), jnp.float32)

    @pl.loop(0, rows)
    def _(i):
        for c in range(cols // 16):
            ref[i, pl.ds(c * 16, 16)] = z16


def _make_deg_kernel(n, nch):
    fl = _flush_rows(n)
    nacc = NS * fl

    @functools.partial(
        pl.kernel,
        out_type=jax.ShapeDtypeStruct((NC, NS, fl), jnp.float32),
        mesh=_mesh(),
        scratch_types=[
            pltpu.VMEM((nch, K), jnp.int32),
            pltpu.VMEM((K,), jnp.float32),
            pltpu.VMEM((fl,), jnp.float32),
            pltpu.VMEM_SHARED((nacc,), jnp.float32),
        ],
        compiler_params=pltpu.CompilerParams(use_tc_tiling_on_sc=False),
    )
    def deg_k(dst_hbm, out_hbm, dst_v, ones_v, zb_v, acc_sh):
        cid = lax.axis_index("c")
        sid = lax.axis_index("s")
        wid = cid * NS + sid
        pltpu.sync_copy(dst_hbm.at[wid], dst_v)

        one16 = jnp.ones((16,), jnp.float32)
        zero16 = jnp.zeros((16,), jnp.float32)

        @pl.loop(0, K // 16)
        def _(i):
            ones_v[pl.ds(i * 16, 16)] = one16

        @pl.loop(0, fl // 16)
        def _(i):
            zb_v[pl.ds(i * 16, 16)] = zero16

        pltpu.sync_copy(zb_v, acc_sh.at[pl.ds(sid * fl, fl)])
        plsc.subcore_barrier()

        @pl.loop(0, nch)
        def _(j):
            pltpu.sync_copy(ones_v, acc_sh.at[dst_v.at[j]], add=True)

        plsc.subcore_barrier()
        pltpu.sync_copy(acc_sh.at[pl.ds(sid * fl, fl)], zb_v)
        pltpu.sync_copy(zb_v, out_hbm.at[cid, sid])

    return deg_k


def _make_prop_kernel(n, c, nch):
    fl = _flush_rows(n)
    nacc = NS * fl

    @functools.partial(
        pl.kernel,
        out_type=jax.ShapeDtypeStruct((NC, NS, fl, c), jnp.float32),
        mesh=_mesh(),
        scratch_types=[
            pltpu.VMEM((nch, K), jnp.int32),
            pltpu.VMEM((nch, K), jnp.int32),
            pltpu.VMEM((NBUF, K, c), jnp.float32),
            pltpu.VMEM((fl, c), jnp.float32),
            pltpu.VMEM_SHARED((nacc, c), jnp.float32),
        ]
        + [pltpu.SemaphoreType.DMA] * NBUF,
        compiler_params=pltpu.CompilerParams(use_tc_tiling_on_sc=False),
    )
    def prop_k(ht_hbm, src_hbm, dst_hbm, out_hbm, src_v, dst_v, ring_v, zb_v,
               acc_sh, *sems):
        cid = lax.axis_index("c")
        sid = lax.axis_index("s")
        wid = cid * NS + sid
        pltpu.sync_copy(src_hbm.at[wid], src_v)
        pltpu.sync_copy(dst_hbm.at[wid], dst_v)

        _zero_fill_2d(zb_v, fl, c)
        pltpu.sync_copy(zb_v, acc_sh.at[pl.ds(sid * fl, fl)])
        plsc.subcore_barrier()

        def gather(j, b):
            return pltpu.make_async_copy(
                ht_hbm.at[src_v.at[j]], ring_v.at[b], sems[b]
            )

        for b in range(NBUF):
            gather(b, b).start()

        @pl.loop(0, (nch - NBUF) // NBUF)
        def _(t):
            go = t * NBUF
            for b in range(NBUF):
                j = go + b
                gather(j, b).wait()
                pltpu.sync_copy(ring_v.at[b], acc_sh.at[dst_v.at[j]], add=True)
                gather(j + NBUF, b).start()

        for b in range(NBUF):
            j = nch - NBUF + b
            gather(j, b).wait()
            pltpu.sync_copy(ring_v.at[b], acc_sh.at[dst_v.at[j]], add=True)

        plsc.subcore_barrier()
        pltpu.sync_copy(acc_sh.at[pl.ds(sid * fl, fl)], zb_v)
        pltpu.sync_copy(zb_v, out_hbm.at[cid, sid])

    return prop_k


def _tc_call(fn, out_shapes, *args):
    return pl.pallas_call(fn, out_shape=out_shapes)(*args)


def _tc1_body(x_ref, w_ref, d0_ref, d1_ref, ht_ref, dinv_ref):
    deg = d0_ref[...] + d1_ref[...] + 1.0
    dinv = 1.0 / jnp.sqrt(deg)
    h = jnp.dot(x_ref[...], w_ref[...], preferred_element_type=jnp.float32)
    ht_ref[...] = h * dinv
    dinv_ref[...] = dinv


def _bn_act(p, g, be, leaky):
    m = jnp.mean(p, axis=0, keepdims=True)
    d = p - m
    v = jnp.mean(d * d, axis=0, keepdims=True)
    y = d * (g / jnp.sqrt(v + 1e-5)) + be
    if leaky:
        return jnp.where(y >= 0.0, y, 0.01 * y)
    return jnp.maximum(y, 0.0)


def _tcb_body(p0_ref, p1_ref, ht_ref, dinv_ref, g_ref, be_ref, out_ref):
    dinv = dinv_ref[...]
    p = dinv * (p0_ref[...] + p1_ref[...] + ht_ref[...])
    y = _bn_act(p, g_ref[...], be_ref[...], leaky=True)
    out_ref[...] = dinv * y


def _tcc_body(p0_ref, p1_ref, ht_ref, dinv_ref, w_ref, g_ref, be_ref, out_ref):
    dinv = dinv_ref[...]
    p = dinv * (p0_ref[...] + p1_ref[...] + ht_ref[...])
    h = jnp.dot(p, w_ref[...], preferred_element_type=jnp.float32)
    y = _bn_act(h, g_ref[...], be_ref[...], leaky=True)
    out_ref[...] = dinv * y


def _tcd_body(p0_ref, p1_ref, ht_ref, dinv_ref, w_ref, g_ref, be_ref, pool_ref,
              out_ref):
    dinv = dinv_ref[...]
    p = dinv * (p0_ref[...] + p1_ref[...] + ht_ref[...])
    h = jnp.dot(p, w_ref[...], preferred_element_type=jnp.float32)
    y = _bn_act(h, g_ref[...], be_ref[...], leaky=False)
    out_ref[...] = jnp.dot(y, pool_ref[...], preferred_element_type=jnp.float32)


def kernel(x, edge_index, W1, b1, g1, be1, W2, b2, g2, be2, W3, b3, g3, be3):
    del b1, b2, b3
    n = x.shape[0]
    e = edge_index.shape[1]
    src = edge_index[0].astype(jnp.int32)
    dst = edge_index[1].astype(jnp.int32)

    ept = -(-e // NW)
    nch = -(-(-(-ept // K)) // NBUF) * NBUF
    ppt = nch * K - ept
    cyc = (jnp.arange(ppt, dtype=jnp.int32) % 128)[None, :].repeat(NW, axis=0)
    srcp = jnp.concatenate(
        [src.reshape(NW, ept), cyc], axis=1).reshape(NW, nch, K)
    dstp = jnp.concatenate(
        [dst.reshape(NW, ept), cyc + n], axis=1).reshape(NW, nch, K)

    fl = _flush_rows(n)

    degp = _make_deg_kernel(n, nch)(dstp)
    degp = degp.reshape(NC, NS * fl)[:, :n]
    deg0 = degp[0].reshape(n, 1)
    deg1 = degp[1].reshape(n, 1)

    c1, c2, c3 = W1.shape[1], W2.shape[1], W3.shape[1]
    f32 = jnp.float32

    ht1, dinv = _tc_call(
        _tc1_body,
        [jax.ShapeDtypeStruct((n, c1), f32), jax.ShapeDtypeStruct((n, 1), f32)],
        x, W1, deg0, deg1,
    )

    def prop(ht, c):
        parts = _make_prop_kernel(n, c, nch)(ht, srcp, dstp)
        parts = parts.reshape(NC, NS * fl, c)[:, :n]
        return parts[0], parts[1]

    p0, p1 = prop(ht1, c1)
    ht2 = _tc_call(
        _tcb_body,
        jax.ShapeDtypeStruct((n, c1), f32),
        p0, p1, ht1, dinv, g1.reshape(1, c1), be1.reshape(1, c1),
    )

    p0, p1 = prop(ht2, c1)
    ht3 = _tc_call(
        _tcc_body,
        jax.ShapeDtypeStruct((n, c2), f32),
        p0, p1, ht2, dinv, W2, g2.reshape(1, c2), be2.reshape(1, c2),
    )

    p0, p1 = prop(ht3, c2)
    pool = np.zeros((c3, c3 // 4), np.float32)
    for i in range(c3):
        pool[i, i // 4] = 0.25
    out = _tc_call(
        _tcd_body,
        jax.ShapeDtypeStruct((n, c3 // 4), f32),
        p0, p1, ht3, dinv, W3, g3.reshape(1, c3), be3.reshape(1, c3),
        jnp.asarray(pool),
    )
    return out


# speedup vs baseline: 49.4514x; 1.1075x over previous
import functools

import jax
import jax.numpy as jnp
import numpy as np
from jax import lax
from jax.experimental import pallas as pl
from jax.experimental.pallas import tpu as pltpu
from jax.experimental.pallas import tpu_sc as plsc

NC = 2
NS = 16
NW = NC * NS
K = 128
NBUF = 4


def _mesh():
    return plsc.VectorSubcoreMesh(
        core_axis_name="c", subcore_axis_name="s", num_cores=NC, num_subcores=NS
    )


def _flush_rows(n):
    return ((n + 128 + 16 * NS - 1) // (16 * NS)) * 16


def _zero_fill_2d(ref, rows, cols):
    z16 = jnp.zeros((16,), jnp.float32)

    @pl.loop(0, rows)
    def _(i):
        for c in range(cols // 16):
            ref[i, pl.ds(c * 16, 16)] = z16


def _make_deg_kernel(n, nch, k):
    fl = _flush_rows(n)
    nacc = NS * fl

    @functools.partial(
        pl.kernel,
        out_type=jax.ShapeDtypeStruct((NC, NS, fl), jnp.float32),
        mesh=_mesh(),
        scratch_types=[
            pltpu.VMEM((nch, k), jnp.int32),
            pltpu.VMEM((K,), jnp.float32),
            pltpu.VMEM((fl,), jnp.float32),
            pltpu.VMEM_SHARED((nacc,), jnp.float32),
        ],
        compiler_params=pltpu.CompilerParams(use_tc_tiling_on_sc=False),
    )
    def deg_k(dst_hbm, out_hbm, dst_v, ones_v, zb_v, acc_sh):
        cid = lax.axis_index("c")
        sid = lax.axis_index("s")
        wid = cid * NS + sid
        pltpu.sync_copy(dst_hbm.at[wid], dst_v)

        one16 = jnp.ones((16,), jnp.float32)
        zero16 = jnp.zeros((16,), jnp.float32)

        @pl.loop(0, K // 16)
        def _(i):
            ones_v[pl.ds(i * 16, 16)] = one16

        @pl.loop(0, fl // 16)
        def _(i):
            zb_v[pl.ds(i * 16, 16)] = zero16

        pltpu.sync_copy(zb_v, acc_sh.at[pl.ds(sid * fl, fl)])
        plsc.subcore_barrier()

        @pl.loop(0, nch)
        def _(j):
            pltpu.sync_copy(ones_v.at[pl.ds(0, k)], acc_sh.at[dst_v.at[j]],
                            add=True)

        plsc.subcore_barrier()
        pltpu.sync_copy(acc_sh.at[pl.ds(sid * fl, fl)], zb_v)
        pltpu.sync_copy(zb_v, out_hbm.at[cid, sid])

    return deg_k


def _make_prop_kernel(n, c, nch, k):
    fl = _flush_rows(n)
    nacc = NS * fl

    @functools.partial(
        pl.kernel,
        out_type=jax.ShapeDtypeStruct((NC, NS, fl, c), jnp.float32),
        mesh=_mesh(),
        scratch_types=[
            pltpu.VMEM((nch, k), jnp.int32),
            pltpu.VMEM((nch, k), jnp.int32),
            pltpu.VMEM((NBUF, k, c), jnp.float32),
            pltpu.VMEM((fl, c), jnp.float32),
            pltpu.VMEM_SHARED((nacc, c), jnp.float32),
        ]
        + [pltpu.SemaphoreType.DMA] * NBUF,
        compiler_params=pltpu.CompilerParams(use_tc_tiling_on_sc=False),
    )
    def prop_k(ht_hbm, src_hbm, dst_hbm, out_hbm, src_v, dst_v, ring_v, zb_v,
               acc_sh, *sems):
        cid = lax.axis_index("c")
        sid = lax.axis_index("s")
        wid = cid * NS + sid
        pltpu.sync_copy(src_hbm.at[wid], src_v)
        pltpu.sync_copy(dst_hbm.at[wid], dst_v)

        _zero_fill_2d(zb_v, fl, c)
        pltpu.sync_copy(zb_v, acc_sh.at[pl.ds(sid * fl, fl)])
        plsc.subcore_barrier()

        def gather(j, b):
            return pltpu.make_async_copy(
                ht_hbm.at[src_v.at[j]], ring_v.at[b], sems[b]
            )

        for b in range(NBUF):
            gather(b, b).start()

        @pl.loop(0, (nch - NBUF) // NBUF)
        def _(t):
            go = t * NBUF
            for b in range(NBUF):
                j = go + b
                gather(j, b).wait()
                pltpu.sync_copy(ring_v.at[b], acc_sh.at[dst_v.at[j]], add=True)
                gather(j + NBUF, b).start()

        for b in range(NBUF):
            j = nch - NBUF + b
            gather(j, b).wait()
            pltpu.sync_copy(ring_v.at[b], acc_sh.at[dst_v.at[j]], add=True)

        plsc.subcore_barrier()
        pltpu.sync_copy(acc_sh.at[pl.ds(sid * fl, fl)], zb_v)
        pltpu.sync_copy(zb_v, out_hbm.at[cid, sid])

    return prop_k


def _tc_call(fn, out_shapes, *args):
    return pl.pallas_call(fn, out_shape=out_shapes)(*args)


def _tc1_body(x_ref, w_ref, d0_ref, d1_ref, ht_ref, dinv_ref):
    deg = d0_ref[...] + d1_ref[...] + 1.0
    dinv = 1.0 / jnp.sqrt(deg)
    h = jnp.dot(x_ref[...], w_ref[...], preferred_element_type=jnp.float32)
    ht_ref[...] = h * dinv
    dinv_ref[...] = dinv


def _bn_act(p, g, be, leaky):
    m = jnp.mean(p, axis=0, keepdims=True)
    d = p - m
    v = jnp.mean(d * d, axis=0, keepdims=True)
    y = d * (g / jnp.sqrt(v + 1e-5)) + be
    if leaky:
        return jnp.where(y >= 0.0, y, 0.01 * y)
    return jnp.maximum(y, 0.0)


def _tcb_body(pall_ref, ht_ref, dinv_ref, g_ref, be_ref, out_ref):
    nn = ht_ref.shape[0]
    dinv = dinv_ref[...]
    p = dinv * (pall_ref[0, :nn, :] + pall_ref[1, :nn, :] + ht_ref[...])
    y = _bn_act(p, g_ref[...], be_ref[...], leaky=True)
    out_ref[...] = dinv * y


def _tcc_body(pall_ref, ht_ref, dinv_ref, w_ref, g_ref, be_ref, out_ref):
    nn = ht_ref.shape[0]
    dinv = dinv_ref[...]
    p = dinv * (pall_ref[0, :nn, :] + pall_ref[1, :nn, :] + ht_ref[...])
    h = jnp.dot(p, w_ref[...], preferred_element_type=jnp.float32)
    y = _bn_act(h, g_ref[...], be_ref[...], leaky=True)
    out_ref[...] = dinv * y


def _tcd_body(pall_ref, ht_ref, dinv_ref, w_ref, g_ref, be_ref, pool_ref,
              out_ref):
    nn = ht_ref.shape[0]
    dinv = dinv_ref[...]
    p = dinv * (pall_ref[0, :nn, :] + pall_ref[1, :nn, :] + ht_ref[...])
    h = jnp.dot(p, w_ref[...], preferred_element_type=jnp.float32)
    y = _bn_act(h, g_ref[...], be_ref[...], leaky=False)
    out_ref[...] = jnp.dot(y, pool_ref[...], preferred_element_type=jnp.float32)


def kernel(x, edge_index, W1, b1, g1, be1, W2, b2, g2, be2, W3, b3, g3, be3):
    del b1, b2, b3
    n = x.shape[0]
    e = edge_index.shape[1]
    src = edge_index[0].astype(jnp.int32)
    dst = edge_index[1].astype(jnp.int32)

    ept = -(-e // NW)
    nch = -(-(-(-ept // K)) // NBUF) * NBUF
    if ept % nch == 0:
        k = ept // nch
        srcp = src.reshape(NW, nch, k)
        dstp = dst.reshape(NW, nch, k)
    else:
        k = K
        ppt = nch * k - ept
        cyc = (jnp.arange(ppt, dtype=jnp.int32) % 128)[None, :].repeat(NW, axis=0)
        srcp = jnp.concatenate(
            [src.reshape(NW, ept), cyc], axis=1).reshape(NW, nch, k)
        dstp = jnp.concatenate(
            [dst.reshape(NW, ept), cyc + n], axis=1).reshape(NW, nch, k)

    fl = _flush_rows(n)

    degp = _make_deg_kernel(n, nch, k)(dstp)
    degp = degp.reshape(NC, NS * fl)[:, :n]
    deg0 = degp[0].reshape(n, 1)
    deg1 = degp[1].reshape(n, 1)

    c1, c2, c3 = W1.shape[1], W2.shape[1], W3.shape[1]
    f32 = jnp.float32

    ht1, dinv = _tc_call(
        _tc1_body,
        [jax.ShapeDtypeStruct((n, c1), f32), jax.ShapeDtypeStruct((n, 1), f32)],
        x, W1, deg0, deg1,
    )

    def prop(ht, c):
        parts = _make_prop_kernel(n, c, nch, k)(ht, srcp, dstp)
        return parts.reshape(NC, NS * fl, c)

    pall = prop(ht1, c1)
    ht2 = _tc_call(
        _tcb_body,
        jax.ShapeDtypeStruct((n, c1), f32),
        pall, ht1, dinv, g1.reshape(1, c1), be1.reshape(1, c1),
    )

    pall = prop(ht2, c1)
    ht3 = _tc_call(
        _tcc_body,
        jax.ShapeDtypeStruct((n, c2), f32),
        pall, ht2, dinv, W2, g2.reshape(1, c2), be2.reshape(1, c2),
    )

    pall = prop(ht3, c2)
    pool = np.zeros((c3, c3 // 4), np.float32)
    for i in range(c3):
        pool[i, i // 4] = 0.25
    out = _tc_call(
        _tcd_body,
        jax.ShapeDtypeStruct((n, c3 // 4), f32),
        pall, ht3, dinv, W3, g3.reshape(1, c3), be3.reshape(1, c3),
        jnp.asarray(pool),
    )
    return out


# speedup vs baseline: 53.2312x; 1.0764x over previous
import functools

import jax
import jax.numpy as jnp
import numpy as np
from jax import lax
from jax.experimental import pallas as pl
from jax.experimental.pallas import tpu as pltpu
from jax.experimental.pallas import tpu_sc as plsc

NC = 2
NS = 16
NW = NC * NS
K = 128
NBUF = 8


def _mesh():
    return plsc.VectorSubcoreMesh(
        core_axis_name="c", subcore_axis_name="s", num_cores=NC, num_subcores=NS
    )


def _flush_rows(n):
    return ((n + 128 + 16 * NS - 1) // (16 * NS)) * 16


def _zero_fill_2d(ref, rows, cols):
    z16 = jnp.zeros((16,), jnp.float32)

    @pl.loop(0, rows)
    def _(i):
        for c in range(cols // 16):
            ref[i, pl.ds(c * 16, 16)] = z16


def _make_deg_kernel(n, nch, k):
    fl = _flush_rows(n)
    nacc = NS * fl

    @functools.partial(
        pl.kernel,
        out_type=jax.ShapeDtypeStruct((NC, NS, fl), jnp.float32),
        mesh=_mesh(),
        scratch_types=[
            pltpu.VMEM((nch, k), jnp.int32),
            pltpu.VMEM((K,), jnp.float32),
            pltpu.VMEM((fl,), jnp.float32),
            pltpu.VMEM_SHARED((nacc,), jnp.float32),
        ],
        compiler_params=pltpu.CompilerParams(use_tc_tiling_on_sc=False),
    )
    def deg_k(dst_hbm, out_hbm, dst_v, ones_v, zb_v, acc_sh):
        cid = lax.axis_index("c")
        sid = lax.axis_index("s")
        wid = cid * NS + sid
        pltpu.sync_copy(dst_hbm.at[wid], dst_v)

        one16 = jnp.ones((16,), jnp.float32)
        zero16 = jnp.zeros((16,), jnp.float32)

        @pl.loop(0, K // 16)
        def _(i):
            ones_v[pl.ds(i * 16, 16)] = one16

        @pl.loop(0, fl // 16)
        def _(i):
            zb_v[pl.ds(i * 16, 16)] = zero16

        pltpu.sync_copy(zb_v, acc_sh.at[pl.ds(sid * fl, fl)])
        plsc.subcore_barrier()

        @pl.loop(0, nch)
        def _(j):
            pltpu.sync_copy(ones_v.at[pl.ds(0, k)], acc_sh.at[dst_v.at[j]],
                            add=True)

        plsc.subcore_barrier()
        pltpu.sync_copy(acc_sh.at[pl.ds(sid * fl, fl)], zb_v)
        pltpu.sync_copy(zb_v, out_hbm.at[cid, sid])

    return deg_k


def _make_prop_kernel(n, c, nch, k):
    fl = _flush_rows(n)
    nacc = NS * fl

    @functools.partial(
        pl.kernel,
        out_type=jax.ShapeDtypeStruct((NC, NS, fl, c), jnp.float32),
        mesh=_mesh(),
        scratch_types=[
            pltpu.VMEM((nch, k), jnp.int32),
            pltpu.VMEM((nch, k), jnp.int32),
            pltpu.VMEM((NBUF, k, c), jnp.float32),
            pltpu.VMEM((fl, c), jnp.float32),
            pltpu.VMEM_SHARED((nacc, c), jnp.float32),
        ]
        + [pltpu.SemaphoreType.DMA] * NBUF,
        compiler_params=pltpu.CompilerParams(use_tc_tiling_on_sc=False),
    )
    def prop_k(ht_hbm, src_hbm, dst_hbm, out_hbm, src_v, dst_v, ring_v, zb_v,
               acc_sh, *sems):
        cid = lax.axis_index("c")
        sid = lax.axis_index("s")
        wid = cid * NS + sid
        pltpu.sync_copy(src_hbm.at[wid], src_v)
        pltpu.sync_copy(dst_hbm.at[wid], dst_v)

        _zero_fill_2d(zb_v, fl, c)
        pltpu.sync_copy(zb_v, acc_sh.at[pl.ds(sid * fl, fl)])
        plsc.subcore_barrier()

        def gather(j, b):
            return pltpu.make_async_copy(
                ht_hbm.at[src_v.at[j]], ring_v.at[b], sems[b]
            )

        for b in range(NBUF):
            gather(b, b).start()

        @pl.loop(0, (nch - NBUF) // NBUF)
        def _(t):
            go = t * NBUF
            for b in range(NBUF):
                j = go + b
                gather(j, b).wait()
                pltpu.sync_copy(ring_v.at[b], acc_sh.at[dst_v.at[j]], add=True)
                gather(j + NBUF, b).start()

        for b in range(NBUF):
            j = nch - NBUF + b
            gather(j, b).wait()
            pltpu.sync_copy(ring_v.at[b], acc_sh.at[dst_v.at[j]], add=True)

        plsc.subcore_barrier()
        pltpu.sync_copy(acc_sh.at[pl.ds(sid * fl, fl)], zb_v)
        pltpu.sync_copy(zb_v, out_hbm.at[cid, sid])

    return prop_k


def _tc_call(fn, out_shapes, *args):
    return pl.pallas_call(fn, out_shape=out_shapes)(*args)


def _tc1_body(x_ref, w_ref, d0_ref, d1_ref, ht_ref, dinv_ref):
    deg = d0_ref[...] + d1_ref[...] + 1.0
    dinv = 1.0 / jnp.sqrt(deg)
    h = jnp.dot(x_ref[...], w_ref[...], preferred_element_type=jnp.float32)
    ht_ref[...] = h * dinv
    dinv_ref[...] = dinv


def _bn_act(p, g, be, leaky):
    m = jnp.mean(p, axis=0, keepdims=True)
    d = p - m
    v = jnp.mean(d * d, axis=0, keepdims=True)
    y = d * (g / jnp.sqrt(v + 1e-5)) + be
    if leaky:
        return jnp.where(y >= 0.0, y, 0.01 * y)
    return jnp.maximum(y, 0.0)


def _tcb_body(pall_ref, ht_ref, dinv_ref, g_ref, be_ref, out_ref):
    nn = ht_ref.shape[0]
    dinv = dinv_ref[...]
    p = dinv * (pall_ref[0, :nn, :] + pall_ref[1, :nn, :] + ht_ref[...])
    y = _bn_act(p, g_ref[...], be_ref[...], leaky=True)
    out_ref[...] = dinv * y


def _tcc_body(pall_ref, ht_ref, dinv_ref, w_ref, g_ref, be_ref, out_ref):
    nn = ht_ref.shape[0]
    dinv = dinv_ref[...]
    p = dinv * (pall_ref[0, :nn, :] + pall_ref[1, :nn, :] + ht_ref[...])
    h = jnp.dot(p, w_ref[...], preferred_element_type=jnp.float32)
    y = _bn_act(h, g_ref[...], be_ref[...], leaky=True)
    out_ref[...] = dinv * y


def _tcd_body(pall_ref, ht_ref, dinv_ref, w_ref, g_ref, be_ref, pool_ref,
              out_ref):
    nn = ht_ref.shape[0]
    dinv = dinv_ref[...]
    p = dinv * (pall_ref[0, :nn, :] + pall_ref[1, :nn, :] + ht_ref[...])
    h = jnp.dot(p, w_ref[...], preferred_element_type=jnp.float32)
    y = _bn_act(h, g_ref[...], be_ref[...], leaky=False)
    out_ref[...] = jnp.dot(y, pool_ref[...], preferred_element_type=jnp.float32)


def kernel(x, edge_index, W1, b1, g1, be1, W2, b2, g2, be2, W3, b3, g3, be3):
    del b1, b2, b3
    n = x.shape[0]
    e = edge_index.shape[1]
    src = edge_index[0].astype(jnp.int32)
    dst = edge_index[1].astype(jnp.int32)

    ept = -(-e // NW)
    nch = -(-(-(-ept // K)) // NBUF) * NBUF
    if ept % nch == 0:
        k = ept // nch
        srcp = src.reshape(NW, nch, k)
        dstp = dst.reshape(NW, nch, k)
    else:
        k = K
        ppt = nch * k - ept
        cyc = (jnp.arange(ppt, dtype=jnp.int32) % 128)[None, :].repeat(NW, axis=0)
        srcp = jnp.concatenate(
            [src.reshape(NW, ept), cyc], axis=1).reshape(NW, nch, k)
        dstp = jnp.concatenate(
            [dst.reshape(NW, ept), cyc + n], axis=1).reshape(NW, nch, k)

    fl = _flush_rows(n)

    degp = _make_deg_kernel(n, nch, k)(dstp)
    degp = degp.reshape(NC, NS * fl)[:, :n]
    deg0 = degp[0].reshape(n, 1)
    deg1 = degp[1].reshape(n, 1)

    c1, c2, c3 = W1.shape[1], W2.shape[1], W3.shape[1]
    f32 = jnp.float32

    ht1, dinv = _tc_call(
        _tc1_body,
        [jax.ShapeDtypeStruct((n, c1), f32), jax.ShapeDtypeStruct((n, 1), f32)],
        x, W1, deg0, deg1,
    )

    def prop(ht, c):
        parts = _make_prop_kernel(n, c, nch, k)(ht, srcp, dstp)
        return parts.reshape(NC, NS * fl, c)

    pall = prop(ht1, c1)
    ht2 = _tc_call(
        _tcb_body,
        jax.ShapeDtypeStruct((n, c1), f32),
        pall, ht1, dinv, g1.reshape(1, c1), be1.reshape(1, c1),
    )

    pall = prop(ht2, c1)
    ht3 = _tc_call(
        _tcc_body,
        jax.ShapeDtypeStruct((n, c2), f32),
        pall, ht2, dinv, W2, g2.reshape(1, c2), be2.reshape(1, c2),
    )

    pall = prop(ht3, c2)
    pool = np.zeros((c3, c3 // 4), np.float32)
    for i in range(c3):
        pool[i, i // 4] = 0.25
    out = _tc_call(
        _tcd_body,
        jax.ShapeDtypeStruct((n, c3 // 4), f32),
        pall, ht3, dinv, W3, g3.reshape(1, c3), be3.reshape(1, c3),
        jnp.asarray(pool),
    )
    return out
